# baseline (speedup 1.0000x reference)
import functools

import jax
import jax.numpy as jnp
from jax import lax
from jax.experimental import pallas as pl
from jax.experimental.pallas import tpu as pltpu
from jax.experimental.pallas import tpu_sc as plsc

VOCAB = 1000000
EMBED_DIM = 64
BATCH = 16384
N_NEG = 20

NC = 2
NS = 16
NW = NC * NS
B_PER_W = BATCH // NW
BC = 32
N_CHUNKS = B_PER_W // BC
NEG_PER_CHUNK = BC * N_NEG

VB = 1024


def _tc_to_linear(w_t):
    grid = (VOCAB + VB - 1) // VB

    def body(in_ref, out_ref):
        x = in_ref[...]
        z = x.T
        out_ref[...] = jnp.concatenate([z[: VB // 2], z[VB // 2:]], axis=1)

    return pl.pallas_call(
        body,
        grid=(grid,),
        in_specs=[pl.BlockSpec((EMBED_DIM, VB), lambda i: (0, i))],
        out_specs=pl.BlockSpec((VB // 2, 128), lambda i: (i, 0)),
        out_shape=jax.ShapeDtypeStruct((VOCAB // 2, 128), jnp.float32),
    )(w_t)


def _sc_partials(word_r, ctx_r, neg_r, ww_lin, wc_lin):
    mesh = plsc.VectorSubcoreMesh(
        core_axis_name="c", subcore_axis_name="s", num_cores=NC,
        num_subcores=NS)

    @functools.partial(
        pl.kernel,
        out_type=[
            jax.ShapeDtypeStruct((BATCH // 8, 128), jnp.float32),
            jax.ShapeDtypeStruct((BATCH // 8, 128), jnp.float32),
        ],
        mesh=mesh,
        compiler_params=pltpu.CompilerParams(use_tc_tiling_on_sc=False),
        scratch_types=[
            pltpu.VMEM((B_PER_W,), jnp.int32),
            pltpu.VMEM((B_PER_W,), jnp.int32),
            pltpu.VMEM((N_NEG, B_PER_W // 128, 128), jnp.int32),
            pltpu.VMEM((BC, EMBED_DIM), jnp.float32),
            pltpu.VMEM((BC, EMBED_DIM), jnp.float32),
            pltpu.VMEM((BC, EMBED_DIM), jnp.float32),
            pltpu.VMEM((BC, EMBED_DIM), jnp.float32),
            pltpu.VMEM((NEG_PER_CHUNK, EMBED_DIM), jnp.float32),
            pltpu.VMEM((NEG_PER_CHUNK, EMBED_DIM), jnp.float32),
            pltpu.VMEM((BC // 8, 128), jnp.float32),
            pltpu.VMEM((BC // 8, 128), jnp.float32),
            pltpu.SemaphoreType.DMA,
            pltpu.SemaphoreType.DMA,
        ],
    )
    def k(word_hbm, ctx_hbm, neg_hbm, ww_hbm, wc_hbm,
          pos_hbm, neg_out_hbm,
          widx, cidx, nidx, wA, wB, cA, cB, nA, nB, pbuf, nbuf,
          semA, semB):
        wid = lax.axis_index("s") * NC + lax.axis_index("c")
        base = wid * B_PER_W
        nrw = B_PER_W // 128

        pltpu.sync_copy(word_hbm.at[pl.ds(base, B_PER_W)], widx)
        pltpu.sync_copy(ctx_hbm.at[pl.ds(base, B_PER_W)], cidx)
        pltpu.sync_copy(neg_hbm.at[:, pl.ds(wid * nrw, nrw)], nidx)

        def fire(ck, w_buf, c_buf, n_buf, sem):
            pltpu.async_copy(
                ww_hbm.at[widx.at[pl.ds(ck * BC, BC)]], w_buf, sem)
            pltpu.async_copy(
                wc_hbm.at[cidx.at[pl.ds(ck * BC, BC)]], c_buf, sem)
            row = ck // 4
            off = (ck % 4) * BC

            def nfire(n, carry):
                pltpu.async_copy(
                    wc_hbm.at[nidx.at[n, row, pl.ds(off, BC)]],
                    n_buf.at[pl.ds(n * BC, BC)], sem)
                return carry

            lax.fori_loop(0, N_NEG, nfire, 0)

        def drain(w_buf, c_buf, n_buf, sem):
            pltpu.make_async_copy(ww_hbm.at[pl.ds(0, BC)], w_buf, sem).wait()
            pltpu.make_async_copy(wc_hbm.at[pl.ds(0, BC)], c_buf, sem).wait()
            pltpu.make_async_copy(
                wc_hbm.at[pl.ds(0, NEG_PER_CHUNK)], n_buf, sem).wait()

        def compute(ck, w_buf, c_buf, n_buf):
            def body_b(b, carry):
                w0 = w_buf[b, pl.ds(0, 16)]
                w1 = w_buf[b, pl.ds(16, 16)]
                w2 = w_buf[b, pl.ds(32, 16)]
                w3 = w_buf[b, pl.ds(48, 16)]
                pos = (w0 * c_buf[b, pl.ds(0, 16)]
                       + w1 * c_buf[b, pl.ds(16, 16)]
                       + w2 * c_buf[b, pl.ds(32, 16)]
                       + w3 * c_buf[b, pl.ds(48, 16)])

                def body_n(n, acc):
                    r = n * BC + b
                    return (acc
                            + w0 * n_buf[r, pl.ds(0, 16)]
                            + w1 * n_buf[r, pl.ds(16, 16)]
                            + w2 * n_buf[r, pl.ds(32, 16)]
                            + w3 * n_buf[r, pl.ds(48, 16)])

                neg = lax.fori_loop(0, N_NEG, body_n,
                                    jnp.zeros((16,), jnp.float32))
                row = b // 8
                off = (b % 8) * 16
                pbuf[row, pl.ds(off, 16)] = pos
                nbuf[row, pl.ds(off, 16)] = neg
                return carry

            lax.fori_loop(0, BC, body_b, 0)
            orow = wid * (B_PER_W // 8) + ck * (BC // 8)
            pltpu.sync_copy(pbuf, pos_hbm.at[pl.ds(orow, BC // 8)])
            pltpu.sync_copy(nbuf, neg_out_hbm.at[pl.ds(orow, BC // 8)])

        fire(0, wA, cA, nA, semA)
        fire(1, wB, cB, nB, semB)

        def loop_body(i, carry):
            ck = 2 * i
            drain(wA, cA, nA, semA)
            compute(ck, wA, cA, nA)
            fire(ck + 2, wA, cA, nA, semA)
            drain(wB, cB, nB, semB)
            compute(ck + 1, wB, cB, nB)
            fire(ck + 3, wB, cB, nB, semB)
            return carry

        lax.fori_loop(0, N_CHUNKS // 2 - 1, loop_body, 0)
        drain(wA, cA, nA, semA)
        compute(N_CHUNKS - 2, wA, cA, nA)
        drain(wB, cB, nB, semB)
        compute(N_CHUNKS - 1, wB, cB, nB)

    return k(word_r, ctx_r, neg_r, ww_lin, wc_lin)


def _tc_loss(pos2, neg2):
    def body(p_ref, n_ref, o_ref):
        p = p_ref[...]
        n = n_ref[...]
        j = lax.broadcasted_iota(jnp.int32, (128, 8), 0)
        k = lax.broadcasted_iota(jnp.int32, (128, 8), 1)
        m = (j // 16 == k).astype(jnp.float32)
        sp = jnp.dot(p, m, preferred_element_type=jnp.float32)
        sn = jnp.dot(n, m, preferred_element_type=jnp.float32)
        l = jax.nn.log_sigmoid(sp) + jax.nn.log_sigmoid(-sn)
        o_ref[...] = (-jnp.sum(l) / BATCH).reshape(1, 1)

    return pl.pallas_call(
        body,
        out_shape=jax.ShapeDtypeStruct((1, 1), jnp.float32),
    )(pos2, neg2)


def _perm(v):
    return v - (v % 1024) + 2 * (v % 512) + ((v >> 9) & 1)


def kernel(word, context, negative, W_word, W_context):
    word_r = _perm(word.astype(jnp.int32))
    ctx_r = _perm(context.astype(jnp.int32))
    neg_r = (_perm(negative.astype(jnp.int32)).T
             .reshape(N_NEG, BATCH // 128, 128))
    ww_lin = _tc_to_linear(W_word.T).reshape(VOCAB, EMBED_DIM)
    wc_lin = _tc_to_linear(W_context.T).reshape(VOCAB, EMBED_DIM)
    pos_part, neg_part = _sc_partials(word_r, ctx_r, neg_r, ww_lin, wc_lin)
    out = _tc_loss(pos_part, neg_part)
    return out.reshape(())


# speedup vs baseline: 3.5675x; 3.5675x over previous
import functools

import jax
import jax.numpy as jnp
from jax import lax
from jax.experimental import pallas as pl
from jax.experimental.pallas import tpu as pltpu
from jax.experimental.pallas import tpu_sc as plsc

VOCAB = 1000000
EMBED_DIM = 64
BATCH = 16384
N_NEG = 20

NC = 2
NS = 16
NW = NC * NS
B_PER_W = BATCH // NW
BC = 32
N_CHUNKS = B_PER_W // BC
NEG_PER_CHUNK = BC * N_NEG

VB = 32768


def _tc_to_linear(w_t):
    grid = (VOCAB + VB - 1) // VB

    def body(in_ref, out_ref):
        x = in_ref[...]
        x2 = jnp.concatenate([x[:, : VB // 2], x[:, VB // 2:]], axis=0)
        out_ref[...] = x2.T

    return pl.pallas_call(
        body,
        grid=(grid,),
        in_specs=[pl.BlockSpec((EMBED_DIM, VB), lambda i: (0, i))],
        out_specs=pl.BlockSpec((VB // 2, 128), lambda i: (i, 0)),
        out_shape=jax.ShapeDtypeStruct((VOCAB // 2, 128), jnp.float32),
    )(w_t)


def _sc_partials(word_r, ctx_r, neg_r, ww_lin, wc_lin):
    mesh = plsc.VectorSubcoreMesh(
        core_axis_name="c", subcore_axis_name="s", num_cores=NC,
        num_subcores=NS)

    @functools.partial(
        pl.kernel,
        out_type=[
            jax.ShapeDtypeStruct((BATCH // 8, 128), jnp.float32),
            jax.ShapeDtypeStruct((BATCH // 8, 128), jnp.float32),
        ],
        mesh=mesh,
        compiler_params=pltpu.CompilerParams(use_tc_tiling_on_sc=False),
        scratch_types=[
            pltpu.VMEM((B_PER_W,), jnp.int32),
            pltpu.VMEM((B_PER_W,), jnp.int32),
            pltpu.VMEM((N_NEG, B_PER_W // 128, 128), jnp.int32),
            pltpu.VMEM((BC, EMBED_DIM), jnp.float32),
            pltpu.VMEM((BC, EMBED_DIM), jnp.float32),
            pltpu.VMEM((BC, EMBED_DIM), jnp.float32),
            pltpu.VMEM((BC, EMBED_DIM), jnp.float32),
            pltpu.VMEM((NEG_PER_CHUNK, EMBED_DIM), jnp.float32),
            pltpu.VMEM((NEG_PER_CHUNK, EMBED_DIM), jnp.float32),
            pltpu.VMEM((BC // 8, 128), jnp.float32),
            pltpu.VMEM((BC // 8, 128), jnp.float32),
            pltpu.SemaphoreType.DMA,
            pltpu.SemaphoreType.DMA,
        ],
    )
    def k(word_hbm, ctx_hbm, neg_hbm, ww_hbm, wc_hbm,
          pos_hbm, neg_out_hbm,
          widx, cidx, nidx, wA, wB, cA, cB, nA, nB, pbuf, nbuf,
          semA, semB):
        wid = lax.axis_index("s") * NC + lax.axis_index("c")
        base = wid * B_PER_W
        nrw = B_PER_W // 128

        pltpu.sync_copy(word_hbm.at[pl.ds(base, B_PER_W)], widx)
        pltpu.sync_copy(ctx_hbm.at[pl.ds(base, B_PER_W)], cidx)
        pltpu.sync_copy(neg_hbm.at[:, pl.ds(wid * nrw, nrw)], nidx)

        def fire(ck, w_buf, c_buf, n_buf, sem):
            pltpu.async_copy(
                ww_hbm.at[widx.at[pl.ds(ck * BC, BC)]], w_buf, sem)
            pltpu.async_copy(
                wc_hbm.at[cidx.at[pl.ds(ck * BC, BC)]], c_buf, sem)
            row = ck // 4
            off = (ck % 4) * BC

            def nfire(n, carry):
                pltpu.async_copy(
                    wc_hbm.at[nidx.at[n, row, pl.ds(off, BC)]],
                    n_buf.at[pl.ds(n * BC, BC)], sem)
                return carry

            lax.fori_loop(0, N_NEG, nfire, 0)

        def drain(w_buf, c_buf, n_buf, sem):
            pltpu.make_async_copy(ww_hbm.at[pl.ds(0, BC)], w_buf, sem).wait()
            pltpu.make_async_copy(wc_hbm.at[pl.ds(0, BC)], c_buf, sem).wait()
            pltpu.make_async_copy(
                wc_hbm.at[pl.ds(0, NEG_PER_CHUNK)], n_buf, sem).wait()

        def compute(ck, w_buf, c_buf, n_buf):
            def body_b(b, carry):
                w0 = w_buf[b, pl.ds(0, 16)]
                w1 = w_buf[b, pl.ds(16, 16)]
                w2 = w_buf[b, pl.ds(32, 16)]
                w3 = w_buf[b, pl.ds(48, 16)]
                pos = (w0 * c_buf[b, pl.ds(0, 16)]
                       + w1 * c_buf[b, pl.ds(16, 16)]
                       + w2 * c_buf[b, pl.ds(32, 16)]
                       + w3 * c_buf[b, pl.ds(48, 16)])

                def body_n(n, acc):
                    r = n * BC + b
                    return (acc
                            + w0 * n_buf[r, pl.ds(0, 16)]
                            + w1 * n_buf[r, pl.ds(16, 16)]
                            + w2 * n_buf[r, pl.ds(32, 16)]
                            + w3 * n_buf[r, pl.ds(48, 16)])

                neg = lax.fori_loop(0, N_NEG, body_n,
                                    jnp.zeros((16,), jnp.float32))
                row = b // 8
                off = (b % 8) * 16
                pbuf[row, pl.ds(off, 16)] = pos
                nbuf[row, pl.ds(off, 16)] = neg
                return carry

            lax.fori_loop(0, BC, body_b, 0)
            orow = wid * (B_PER_W // 8) + ck * (BC // 8)
            pltpu.sync_copy(pbuf, pos_hbm.at[pl.ds(orow, BC // 8)])
            pltpu.sync_copy(nbuf, neg_out_hbm.at[pl.ds(orow, BC // 8)])

        fire(0, wA, cA, nA, semA)
        fire(1, wB, cB, nB, semB)

        def loop_body(i, carry):
            ck = 2 * i
            drain(wA, cA, nA, semA)
            compute(ck, wA, cA, nA)
            fire(ck + 2, wA, cA, nA, semA)
            drain(wB, cB, nB, semB)
            compute(ck + 1, wB, cB, nB)
            fire(ck + 3, wB, cB, nB, semB)
            return carry

        lax.fori_loop(0, N_CHUNKS // 2 - 1, loop_body, 0)
        drain(wA, cA, nA, semA)
        compute(N_CHUNKS - 2, wA, cA, nA)
        drain(wB, cB, nB, semB)
        compute(N_CHUNKS - 1, wB, cB, nB)

    return k(word_r, ctx_r, neg_r, ww_lin, wc_lin)


def _tc_loss(pos2, neg2):
    def body(p_ref, n_ref, o_ref):
        p = p_ref[...]
        n = n_ref[...]
        j = lax.broadcasted_iota(jnp.int32, (128, 8), 0)
        k = lax.broadcasted_iota(jnp.int32, (128, 8), 1)
        m = (j // 16 == k).astype(jnp.float32)
        sp = jnp.dot(p, m, preferred_element_type=jnp.float32)
        sn = jnp.dot(n, m, preferred_element_type=jnp.float32)
        l = jax.nn.log_sigmoid(sp) + jax.nn.log_sigmoid(-sn)
        o_ref[...] = (-jnp.sum(l) / BATCH).reshape(1, 1)

    return pl.pallas_call(
        body,
        out_shape=jax.ShapeDtypeStruct((1, 1), jnp.float32),
    )(pos2, neg2)


def _perm(v):
    return v - (v % VB) + 2 * (v % (VB // 2)) + ((v // (VB // 2)) & 1)


def kernel(word, context, negative, W_word, W_context):
    word_r = _perm(word.astype(jnp.int32))
    ctx_r = _perm(context.astype(jnp.int32))
    neg_r = (_perm(negative.astype(jnp.int32)).T
             .reshape(N_NEG, BATCH // 128, 128))
    ww_lin = _tc_to_linear(W_word.T).reshape(VOCAB, EMBED_DIM)
    wc_lin = _tc_to_linear(W_context.T).reshape(VOCAB, EMBED_DIM)
    pos_part, neg_part = _sc_partials(word_r, ctx_r, neg_r, ww_lin, wc_lin)
    out = _tc_loss(pos_part, neg_part)
    return out.reshape(())


# speedup vs baseline: 3.7895x; 1.0622x over previous
import functools

import jax
import jax.numpy as jnp
from jax import lax
from jax.experimental import pallas as pl
from jax.experimental.pallas import tpu as pltpu
from jax.experimental.pallas import tpu_sc as plsc

VOCAB = 1000000
EMBED_DIM = 64
BATCH = 16384
N_NEG = 20

NC = 2
NS = 16
NW = NC * NS
B_PER_W = BATCH // NW
BC = 32
N_CHUNKS = B_PER_W // BC
NEG_PER_CHUNK = BC * N_NEG

VB = 32768


def _tc_to_linear(w_t):
    grid = (VOCAB + VB - 1) // VB

    def body(in_ref, out_ref):
        x = in_ref[...]
        x2 = jnp.concatenate([x[:, : VB // 2], x[:, VB // 2:]], axis=0)
        out_ref[...] = x2.T

    return pl.pallas_call(
        body,
        grid=(grid,),
        in_specs=[pl.BlockSpec((EMBED_DIM, VB), lambda i: (0, i))],
        out_specs=pl.BlockSpec((VB // 2, 128), lambda i: (i, 0)),
        out_shape=jax.ShapeDtypeStruct((VOCAB // 2, 128), jnp.float32),
    )(w_t)


def _sc_mesh():
    return plsc.VectorSubcoreMesh(
        core_axis_name="c", subcore_axis_name="s", num_cores=NC,
        num_subcores=NS)


def _sc_negsum(neg_r, wc_lin):
    @functools.partial(
        pl.kernel,
        out_type=jax.ShapeDtypeStruct((BATCH, EMBED_DIM), jnp.float32),
        mesh=_sc_mesh(),
        compiler_params=pltpu.CompilerParams(use_tc_tiling_on_sc=False),
        scratch_types=[
            pltpu.VMEM((N_NEG, B_PER_W // 128, 128), jnp.int32),
            pltpu.VMEM((NEG_PER_CHUNK, EMBED_DIM), jnp.float32),
            pltpu.VMEM((NEG_PER_CHUNK, EMBED_DIM), jnp.float32),
            pltpu.VMEM((BC, EMBED_DIM), jnp.float32),
            pltpu.SemaphoreType.DMA,
            pltpu.SemaphoreType.DMA,
        ],
    )
    def k(neg_hbm, wc_hbm, s_hbm, nidx, nA, nB, sbuf, semA, semB):
        wid = lax.axis_index("s") * NC + lax.axis_index("c")
        base = wid * B_PER_W
        nrw = B_PER_W // 128

        pltpu.sync_copy(neg_hbm.at[:, pl.ds(wid * nrw, nrw)], nidx)

        def fire(ck, n_buf, sem):
            row = ck // 4
            off = (ck % 4) * BC

            def nfire(n, carry):
                pltpu.async_copy(
                    wc_hbm.at[nidx.at[n, row, pl.ds(off, BC)]],
                    n_buf.at[pl.ds(n * BC, BC)], sem)
                return carry

            lax.fori_loop(0, N_NEG, nfire, 0)

        def drain(n_buf, sem):
            pltpu.make_async_copy(
                wc_hbm.at[pl.ds(0, NEG_PER_CHUNK)], n_buf, sem).wait()

        def compute(ck, n_buf):
            def body_b(b, carry):
                def body_n(n, accs):
                    a0, a1, a2, a3 = accs
                    r = n * BC + b
                    return (a0 + n_buf[r, pl.ds(0, 16)],
                            a1 + n_buf[r, pl.ds(16, 16)],
                            a2 + n_buf[r, pl.ds(32, 16)],
                            a3 + n_buf[r, pl.ds(48, 16)])

                z = jnp.zeros((16,), jnp.float32)
                s0, s1, s2, s3 = lax.fori_loop(0, N_NEG, body_n,
                                               (z, z, z, z))
                sbuf[b, pl.ds(0, 16)] = s0
                sbuf[b, pl.ds(16, 16)] = s1
                sbuf[b, pl.ds(32, 16)] = s2
                sbuf[b, pl.ds(48, 16)] = s3
                return carry

            lax.fori_loop(0, BC, body_b, 0)
            pltpu.sync_copy(sbuf, s_hbm.at[pl.ds(base + ck * BC, BC)])

        fire(0, nA, semA)
        fire(1, nB, semB)

        def loop_body(i, carry):
            ck = 2 * i
            drain(nA, semA)
            compute(ck, nA)
            fire(ck + 2, nA, semA)
            drain(nB, semB)
            compute(ck + 1, nB)
            fire(ck + 3, nB, semB)
            return carry

        lax.fori_loop(0, N_CHUNKS // 2 - 1, loop_body, 0)
        drain(nA, semA)
        compute(N_CHUNKS - 2, nA)
        drain(nB, semB)
        compute(N_CHUNKS - 1, nB)

    return k(neg_r, wc_lin)


def _sc_dots(word_r, ctx_r, ww_lin, wc_lin, srow):
    @functools.partial(
        pl.kernel,
        out_type=[
            jax.ShapeDtypeStruct((BATCH // 8, 128), jnp.float32),
            jax.ShapeDtypeStruct((BATCH // 8, 128), jnp.float32),
        ],
        mesh=_sc_mesh(),
        compiler_params=pltpu.CompilerParams(use_tc_tiling_on_sc=False),
        scratch_types=[
            pltpu.VMEM((B_PER_W,), jnp.int32),
            pltpu.VMEM((B_PER_W,), jnp.int32),
            pltpu.VMEM((BC, EMBED_DIM), jnp.float32),
            pltpu.VMEM((BC, EMBED_DIM), jnp.float32),
            pltpu.VMEM((BC, EMBED_DIM), jnp.float32),
            pltpu.VMEM((BC, EMBED_DIM), jnp.float32),
            pltpu.VMEM((BC, EMBED_DIM), jnp.float32),
            pltpu.VMEM((BC, EMBED_DIM), jnp.float32),
            pltpu.VMEM((BC // 8, 128), jnp.float32),
            pltpu.VMEM((BC // 8, 128), jnp.float32),
            pltpu.SemaphoreType.DMA,
            pltpu.SemaphoreType.DMA,
        ],
    )
    def k(word_hbm, ctx_hbm, ww_hbm, wc_hbm, s_hbm,
          pos_hbm, neg_out_hbm,
          widx, cidx, wA, wB, cA, cB, sA, sB, pbuf, nbuf,
          semA, semB):
        wid = lax.axis_index("s") * NC + lax.axis_index("c")
        base = wid * B_PER_W

        pltpu.sync_copy(word_hbm.at[pl.ds(base, B_PER_W)], widx)
        pltpu.sync_copy(ctx_hbm.at[pl.ds(base, B_PER_W)], cidx)

        def fire(ck, w_buf, c_buf, s_buf, sem):
            pltpu.async_copy(
                ww_hbm.at[widx.at[pl.ds(ck * BC, BC)]], w_buf, sem)
            pltpu.async_copy(
                wc_hbm.at[cidx.at[pl.ds(ck * BC, BC)]], c_buf, sem)
            pltpu.async_copy(
                s_hbm.at[pl.ds(base + ck * BC, BC)], s_buf, sem)

        def drain(w_buf, c_buf, s_buf, sem):
            pltpu.make_async_copy(ww_hbm.at[pl.ds(0, BC)], w_buf, sem).wait()
            pltpu.make_async_copy(wc_hbm.at[pl.ds(0, BC)], c_buf, sem).wait()
            pltpu.make_async_copy(s_hbm.at[pl.ds(0, BC)], s_buf, sem).wait()

        def compute(ck, w_buf, c_buf, s_buf):
            def body_b(b, carry):
                w0 = w_buf[b, pl.ds(0, 16)]
                w1 = w_buf[b, pl.ds(16, 16)]
                w2 = w_buf[b, pl.ds(32, 16)]
                w3 = w_buf[b, pl.ds(48, 16)]
                pos = (w0 * c_buf[b, pl.ds(0, 16)]
                       + w1 * c_buf[b, pl.ds(16, 16)]
                       + w2 * c_buf[b, pl.ds(32, 16)]
                       + w3 * c_buf[b, pl.ds(48, 16)])
                neg = (w0 * s_buf[b, pl.ds(0, 16)]
                       + w1 * s_buf[b, pl.ds(16, 16)]
                       + w2 * s_buf[b, pl.ds(32, 16)]
                       + w3 * s_buf[b, pl.ds(48, 16)])
                row = b // 8
                off = (b % 8) * 16
                pbuf[row, pl.ds(off, 16)] = pos
                nbuf[row, pl.ds(off, 16)] = neg
                return carry

            lax.fori_loop(0, BC, body_b, 0)
            orow = wid * (B_PER_W // 8) + ck * (BC // 8)
            pltpu.sync_copy(pbuf, pos_hbm.at[pl.ds(orow, BC // 8)])
            pltpu.sync_copy(nbuf, neg_out_hbm.at[pl.ds(orow, BC // 8)])

        fire(0, wA, cA, sA, semA)
        fire(1, wB, cB, sB, semB)

        def loop_body(i, carry):
            ck = 2 * i
            drain(wA, cA, sA, semA)
            compute(ck, wA, cA, sA)
            fire(ck + 2, wA, cA, sA, semA)
            drain(wB, cB, sB, semB)
            compute(ck + 1, wB, cB, sB)
            fire(ck + 3, wB, cB, sB, semB)
            return carry

        lax.fori_loop(0, N_CHUNKS // 2 - 1, loop_body, 0)
        drain(wA, cA, sA, semA)
        compute(N_CHUNKS - 2, wA, cA, sA)
        drain(wB, cB, sB, semB)
        compute(N_CHUNKS - 1, wB, cB, sB)

    return k(word_r, ctx_r, ww_lin, wc_lin, srow)


def _tc_loss(pos2, neg2):
    def body(p_ref, n_ref, o_ref):
        p = p_ref[...]
        n = n_ref[...]
        j = lax.broadcasted_iota(jnp.int32, (128, 8), 0)
        k = lax.broadcasted_iota(jnp.int32, (128, 8), 1)
        m = (j // 16 == k).astype(jnp.float32)
        sp = jnp.dot(p, m, preferred_element_type=jnp.float32)
        sn = jnp.dot(n, m, preferred_element_type=jnp.float32)
        l = jax.nn.log_sigmoid(sp) + jax.nn.log_sigmoid(-sn)
        o_ref[...] = (-jnp.sum(l) / BATCH).reshape(1, 1)

    return pl.pallas_call(
        body,
        out_shape=jax.ShapeDtypeStruct((1, 1), jnp.float32),
    )(pos2, neg2)


def _perm(v):
    return v - (v % VB) + 2 * (v % (VB // 2)) + ((v // (VB // 2)) & 1)


def kernel(word, context, negative, W_word, W_context):
    word_r = _perm(word.astype(jnp.int32))
    ctx_r = _perm(context.astype(jnp.int32))
    neg_r = (_perm(negative.astype(jnp.int32)).T
             .reshape(N_NEG, BATCH // 128, 128))
    wc_lin = _tc_to_linear(W_context.T).reshape(VOCAB, EMBED_DIM)
    srow = _sc_negsum(neg_r, wc_lin)
    ww_lin = _tc_to_linear(W_word.T).reshape(VOCAB, EMBED_DIM)
    pos_part, neg_part = _sc_dots(word_r, ctx_r, ww_lin, wc_lin, srow)
    out = _tc_loss(pos_part, neg_part)
    return out.reshape(())
